# baseline (speedup 1.0000x reference)
import jax
import jax.numpy as jnp
from jax.experimental import pallas as pl
from jax.experimental.pallas import tpu as pltpu

B = 1024
FT_IN = 64 * 64 * 10 + 64 * 10
K_HALF = 256
NB = K_HALF + 1
BK = 1664
WCHUNK = 5

_DN = (((1,), (1,)), ((), ()))


def _nnue_kernel(x1_ref, x2_ref, w_ref, b_ref, h1w_ref, h1b_ref, h2w_ref,
                 h2b_ref, outw_ref, outb_ref, out_ref, acc1, acc2):
    k = pl.program_id(0)

    @pl.when(k == 0)
    def _init():
        acc1[...] = jnp.zeros_like(acc1)
        acc2[...] = jnp.zeros_like(acc2)

    w = w_ref[:, pl.ds((k % WCHUNK) * BK, BK)]
    acc1[...] += jax.lax.dot_general(x1_ref[...], w, _DN,
                                     preferred_element_type=jnp.float32)
    acc2[...] += jax.lax.dot_general(x2_ref[...], w, _DN,
                                     preferred_element_type=jnp.float32)

    @pl.when(k == pl.num_programs(0) - 1)
    def _tail():
        b = b_ref[...]
        a1 = acc1[...] + b
        a2 = acc2[...] + b
        f1 = jnp.clip(a1[:, :K_HALF], 0.0, 1.0)
        f2 = jnp.clip(a2[:, :K_HALF], 0.0, 1.0)
        psqt = 0.5 * (a1[:, K_HALF:NB] - a2[:, K_HALF:NB])
        h1w = h1w_ref[...]
        h1 = (jax.lax.dot_general(f1, h1w[:, :K_HALF], _DN,
                                  preferred_element_type=jnp.float32)
              + jax.lax.dot_general(f2, h1w[:, K_HALF:], _DN,
                                    preferred_element_type=jnp.float32)
              + h1b_ref[...])
        h1 = jnp.clip(h1, 0.0, 1.0)
        h2 = jax.lax.dot_general(h1, h2w_ref[...], _DN,
                                 preferred_element_type=jnp.float32)
        h2 = jnp.clip(h2 + h2b_ref[...], 0.0, 1.0)
        out = jnp.sum(h2 * outw_ref[...], axis=1, keepdims=True)
        out_ref[...] = out + outb_ref[0] + 16.0 * psqt / 64.0


def kernel(features1, features2, ft_w, ft_b, h1_w, h1_b, h2_w, h2_b,
           out_w, out_b):
    grid = (FT_IN // BK,)
    return pl.pallas_call(
        _nnue_kernel,
        grid=grid,
        in_specs=[
            pl.BlockSpec((B, BK), lambda k: (0, k)),
            pl.BlockSpec((B, BK), lambda k: (0, k)),
            pl.BlockSpec((NB, WCHUNK * BK), lambda k: (0, k // WCHUNK)),
            pl.BlockSpec((1, NB), lambda k: (0, 0)),
            pl.BlockSpec((32, 2 * K_HALF), lambda k: (0, 0)),
            pl.BlockSpec((1, 32), lambda k: (0, 0)),
            pl.BlockSpec((32, 32), lambda k: (0, 0)),
            pl.BlockSpec((1, 32), lambda k: (0, 0)),
            pl.BlockSpec((1, 32), lambda k: (0, 0)),
            pl.BlockSpec(memory_space=pltpu.SMEM),
        ],
        out_specs=pl.BlockSpec((B, 1), lambda k: (0, 0)),
        out_shape=jax.ShapeDtypeStruct((B, 1), jnp.float32),
        scratch_shapes=[
            pltpu.VMEM((B, NB), jnp.float32),
            pltpu.VMEM((B, NB), jnp.float32),
        ],
        compiler_params=pltpu.CompilerParams(
            dimension_semantics=("arbitrary",),
            vmem_limit_bytes=120 * 1024 * 1024),
    )(features1, features2, ft_w, ft_b.reshape(1, NB), h1_w,
      h1_b.reshape(1, 32), h2_w, h2_b.reshape(1, 32), out_w, out_b)


# speedup vs baseline: 1.0752x; 1.0752x over previous
import jax
import jax.numpy as jnp
from jax.experimental import pallas as pl
from jax.experimental.pallas import tpu as pltpu

B = 1024
FT_IN = 64 * 64 * 10 + 64 * 10
K_HALF = 256
NB = K_HALF + 1
BK = 3200

_DN = (((1,), (1,)), ((), ()))


def _nnue_kernel(x1_ref, x2_ref, w_ref, b_ref, h1w_ref, h1b_ref, h2w_ref,
                 h2b_ref, outw_ref, outb_ref, out_ref, acc1, acc2):
    k = pl.program_id(0)

    @pl.when(k == 0)
    def _init():
        acc1[...] = jnp.zeros_like(acc1)
        acc2[...] = jnp.zeros_like(acc2)

    w = w_ref[...]
    acc1[...] += jax.lax.dot_general(x1_ref[...], w, _DN,
                                     preferred_element_type=jnp.float32)
    acc2[...] += jax.lax.dot_general(x2_ref[...], w, _DN,
                                     preferred_element_type=jnp.float32)

    @pl.when(k == pl.num_programs(0) - 1)
    def _tail():
        b = b_ref[...]
        a1 = acc1[...] + b
        a2 = acc2[...] + b
        f1 = jnp.clip(a1[:, :K_HALF], 0.0, 1.0)
        f2 = jnp.clip(a2[:, :K_HALF], 0.0, 1.0)
        psqt = 0.5 * (a1[:, K_HALF:NB] - a2[:, K_HALF:NB])
        h1w = h1w_ref[...]
        h1 = (jax.lax.dot_general(f1, h1w[:, :K_HALF], _DN,
                                  preferred_element_type=jnp.float32)
              + jax.lax.dot_general(f2, h1w[:, K_HALF:], _DN,
                                    preferred_element_type=jnp.float32)
              + h1b_ref[...])
        h1 = jnp.clip(h1, 0.0, 1.0)
        h2 = jax.lax.dot_general(h1, h2w_ref[...], _DN,
                                 preferred_element_type=jnp.float32)
        h2 = jnp.clip(h2 + h2b_ref[...], 0.0, 1.0)
        out = jnp.sum(h2 * outw_ref[...], axis=1, keepdims=True)
        out_ref[...] = out + outb_ref[0] + 16.0 * psqt / 64.0


def kernel(features1, features2, ft_w, ft_b, h1_w, h1_b, h2_w, h2_b,
           out_w, out_b):
    grid = (FT_IN // BK,)
    return pl.pallas_call(
        _nnue_kernel,
        grid=grid,
        in_specs=[
            pl.BlockSpec((B, BK), lambda k: (0, k)),
            pl.BlockSpec((B, BK), lambda k: (0, k)),
            pl.BlockSpec((NB, BK), lambda k: (0, k)),
            pl.BlockSpec((1, NB), lambda k: (0, 0)),
            pl.BlockSpec((32, 2 * K_HALF), lambda k: (0, 0)),
            pl.BlockSpec((1, 32), lambda k: (0, 0)),
            pl.BlockSpec((32, 32), lambda k: (0, 0)),
            pl.BlockSpec((1, 32), lambda k: (0, 0)),
            pl.BlockSpec((1, 32), lambda k: (0, 0)),
            pl.BlockSpec(memory_space=pltpu.SMEM),
        ],
        out_specs=pl.BlockSpec((B, 1), lambda k: (0, 0)),
        out_shape=jax.ShapeDtypeStruct((B, 1), jnp.float32),
        scratch_shapes=[
            pltpu.VMEM((B, NB), jnp.float32),
            pltpu.VMEM((B, NB), jnp.float32),
        ],
        compiler_params=pltpu.CompilerParams(
            dimension_semantics=("arbitrary",),
            vmem_limit_bytes=120 * 1024 * 1024),
    )(features1, features2, ft_w, ft_b.reshape(1, NB), h1_w,
      h1_b.reshape(1, 32), h2_w, h2_b.reshape(1, 32), out_w, out_b)
